# baseline (speedup 1.0000x reference)
import jax
import jax.numpy as jnp
from jax import lax
from jax.experimental import pallas as pl
from jax.experimental.pallas import tpu as pltpu
from jax.experimental.pallas import tpu_sc as plsc

N = 10000
D = 128
E = 320000
NC = 2
NS = 16
NW = NC * NS
KE = 128
ND = 80
NCH = 4
EPW = ND * KE
EPAD = NW * EPW
NP = 10240
RPT = NP // NS
GS = 2
SLOPE = 0.2
_f32 = jnp.float32


def _leaky(v):
    return jnp.where(v > 0, v, SLOPE * v)


def _dense1_body(x_ref, W_ref, b_ref, a_ref, h_ref, s_ref, t_ref, smax_ref):
    h = jnp.dot(x_ref[...], W_ref[...], preferred_element_type=jnp.float32)
    h = _leaky(h + b_ref[...][None, :])
    h_ref[...] = h
    av = a_ref[...][:, 0]
    s = jnp.sum(h * av[:D][None, :], axis=1)
    t = jnp.sum(h * av[D:][None, :], axis=1)
    s_ref[...] = s
    t_ref[...] = t
    smax_ref[...] = jnp.broadcast_to(jnp.max(s), (16,))


def _combine_dense_body(Hp_ref, dp_ref, W_ref, b_ref, a_ref,
                        hp_ref, h_ref, s_ref, t_ref, smax_ref):
    Hs = Hp_ref[0, :N, :] + Hp_ref[1, :N, :]
    dn = dp_ref[0, :N] + dp_ref[1, :N]
    dn = jnp.where(dn == 0, 1.0, dn)
    hp = Hs / dn[:, None]
    hp_ref[...] = hp
    h = jnp.dot(hp, W_ref[...], preferred_element_type=jnp.float32)
    h = _leaky(h + b_ref[...][None, :])
    h_ref[...] = h
    av = a_ref[...][:, 0]
    s = jnp.sum(h * av[:D][None, :], axis=1)
    t = jnp.sum(h * av[D:][None, :], axis=1)
    s_ref[...] = s
    t_ref[...] = t
    smax_ref[...] = jnp.broadcast_to(jnp.max(s), (16,))


def _final_body(h1p_ref, Hp_ref, dp_ref, out_ref):
    Hs = Hp_ref[0, :N, :] + Hp_ref[1, :N, :]
    dn = dp_ref[0, :N] + dp_ref[1, :N]
    dn = jnp.where(dn == 0, 1.0, dn)
    out_ref[:, :D] = h1p_ref[...]
    out_ref[:, D:] = Hs / dn[:, None]


def _dense1(x, W1, b1, a):
    return pl.pallas_call(
        _dense1_body,
        out_shape=(
            jax.ShapeDtypeStruct((N, D), _f32),
            jax.ShapeDtypeStruct((N,), _f32),
            jax.ShapeDtypeStruct((N,), _f32),
            jax.ShapeDtypeStruct((16,), _f32),
        ),
    )(x, W1, b1, a)


def _combine_dense(Hp, dp, W2, b2, a):
    return pl.pallas_call(
        _combine_dense_body,
        out_shape=(
            jax.ShapeDtypeStruct((N, D), _f32),
            jax.ShapeDtypeStruct((N, D), _f32),
            jax.ShapeDtypeStruct((N,), _f32),
            jax.ShapeDtypeStruct((N,), _f32),
            jax.ShapeDtypeStruct((16,), _f32),
        ),
    )(Hp, dp, W2, b2, a)


def _final(h1p, Hp, dp):
    return pl.pallas_call(
        _final_body,
        out_shape=jax.ShapeDtypeStruct((N, 2 * D), _f32),
    )(h1p, Hp, dp)


def _wts_body(s_hbm, t_hbm, smax_hbm, src_hbm, dst_hbm, w_hbm,
              s_v, t_v, smax_v, src_v, dst_v, w_v):
    c = lax.axis_index("c")
    sub = lax.axis_index("s")
    wid = c * NS + sub

    pltpu.sync_copy(s_hbm, s_v)
    pltpu.sync_copy(t_hbm, t_v)
    pltpu.sync_copy(smax_hbm, smax_v)
    smax = smax_v[...]
    base0 = wid * EPW

    def _chunk(ch, _):
        pltpu.sync_copy(src_hbm.at[wid, pl.ds(ch * NCH, NCH)], src_v)
        pltpu.sync_copy(dst_hbm.at[wid, pl.ds(ch * NCH, NCH)], dst_v)
        for jj in range(NCH):
            valid = jnp.where(
                base0 + (ch * NCH + jj) * KE < E, 1.0, 0.0).astype(_f32)
            for g in range(KE // 16):
                sidx = src_v[jj, pl.ds(g * 16, 16)]
                didx = dst_v[jj, pl.ds(g * 16, 16)]
                sg = plsc.load_gather(s_v, [sidx])
                tg = plsc.load_gather(t_v, [didx])
                e = _leaky(sg + tg)
                cc = _leaky(smax + tg)
                w_v[jj, pl.ds(g * 16, 16)] = jnp.exp(e - cc) * valid
        pltpu.sync_copy(w_v, w_hbm.at[wid, pl.ds(ch * NCH, NCH)])
        return 0

    lax.fori_loop(0, ND // NCH, _chunk, 0)


_wts_pass = pl.kernel(
    _wts_body,
    out_type=jax.ShapeDtypeStruct((NW, ND, KE), _f32),
    mesh=plsc.VectorSubcoreMesh(core_axis_name="c", subcore_axis_name="s",
                                num_cores=NC, num_subcores=NS),
    compiler_params=pltpu.CompilerParams(needs_layout_passes=False),
    scratch_types=[
        pltpu.VMEM((N,), _f32),
        pltpu.VMEM((N,), _f32),
        pltpu.VMEM((16,), _f32),
        pltpu.VMEM((NCH, KE), jnp.int32),
        pltpu.VMEM((NCH, KE), jnp.int32),
        pltpu.VMEM((NCH, KE), _f32),
    ],
)


def _agg_body(h_hbm, src_hbm, dst_hbm, w_hbm,
              Hout, dout,
              src_v, dst_v, w_v, rows_v, zd_v,
              H_sh, d_sh, sem_g, sem_s):
    c = lax.axis_index("c")
    sub = lax.axis_index("s")
    wid = c * NS + sub

    z16 = jnp.zeros((16,), _f32)

    def _zb(i, _):
        r = i // 8
        q = i % 8
        rows_v[0, r, pl.ds(q * 16, 16)] = z16
        return 0
    lax.fori_loop(0, KE * 8, _zb, 0)

    def _zd0(i, _):
        zd_v[pl.ds(i * 16, 16)] = z16
        return 0
    lax.fori_loop(0, RPT // 16, _zd0, 0)

    def _zH(i, _):
        pltpu.sync_copy(rows_v.at[0], H_sh.at[pl.ds(sub * RPT + i * KE, KE)])
        return 0
    lax.fori_loop(0, RPT // KE, _zH, 0)

    pltpu.sync_copy(zd_v, d_sh.at[pl.ds(sub * RPT, RPT)])

    plsc.subcore_barrier()

    pltpu.sync_copy(src_hbm.at[wid, pl.ds(0, NCH)], src_v.at[0])
    pltpu.sync_copy(dst_hbm.at[wid, pl.ds(0, NCH)], dst_v.at[0])
    pltpu.sync_copy(w_hbm.at[wid, pl.ds(0, NCH)], w_v.at[0])
    for k in range(GS - 1):
        pltpu.async_copy(h_hbm.at[src_v.at[0, k]], rows_v.at[k], sem_g.at[k])

    def _step(j, _):
        p = j % GS
        csl = (j // NCH) % 2
        jj = j % NCH

        pltpu.make_async_copy(h_hbm.at[src_v.at[csl, jj]], rows_v.at[p],
                              sem_g.at[p]).wait()

        for g in range(KE // 16):
            wg = w_v[csl, jj, pl.ds(g * 16, 16)]
            for r in range(16):
                row = g * 16 + r
                spl = jnp.take_along_axis(
                    wg, jnp.full((16,), r, jnp.int32),
                    axis=0, mode="promise_in_bounds")
                for q in range(D // 16):
                    rows_v[p, row, pl.ds(q * 16, 16)] = (
                        rows_v[p, row, pl.ds(q * 16, 16)] * spl)

        @pl.when(j >= 1)
        def _():
            jm = j - 1
            pm = jm % GS
            cslm = (jm // NCH) % 2
            jjm = jm % NCH
            pltpu.make_async_copy(rows_v.at[pm],
                                  H_sh.at[dst_v.at[cslm, jjm]],
                                  sem_s.at[pm]).wait()
            pltpu.make_async_copy(w_v.at[cslm, jjm],
                                  d_sh.at[dst_v.at[cslm, jjm]],
                                  sem_s.at[pm]).wait()

        pltpu.async_copy(rows_v.at[p], H_sh.at[dst_v.at[csl, jj]],
                         sem_s.at[p], add=True)
        pltpu.async_copy(w_v.at[csl, jj], d_sh.at[dst_v.at[csl, jj]],
                         sem_s.at[p], add=True)

        @pl.when(j + GS - 1 < ND)
        def _():
            j2 = j + GS - 1
            p2 = j2 % GS
            ch2 = j2 // NCH
            csl2 = ch2 % 2
            jj2 = j2 % NCH

            @pl.when(jj2 == 0)
            def _():
                pltpu.sync_copy(src_hbm.at[wid, pl.ds(ch2 * NCH, NCH)],
                                src_v.at[csl2])
                pltpu.sync_copy(dst_hbm.at[wid, pl.ds(ch2 * NCH, NCH)],
                                dst_v.at[csl2])
                pltpu.sync_copy(w_hbm.at[wid, pl.ds(ch2 * NCH, NCH)],
                                w_v.at[csl2])

            pltpu.async_copy(h_hbm.at[src_v.at[csl2, jj2]], rows_v.at[p2],
                             sem_g.at[p2])
        return 0

    lax.fori_loop(0, ND, _step, 0)

    jf = ND - 1
    pf = jf % GS
    cslf = (jf // NCH) % 2
    jjf = jf % NCH
    pltpu.make_async_copy(rows_v.at[pf], H_sh.at[dst_v.at[cslf, jjf]],
                          sem_s.at[pf]).wait()
    pltpu.make_async_copy(w_v.at[cslf, jjf], d_sh.at[dst_v.at[cslf, jjf]],
                          sem_s.at[pf]).wait()

    plsc.subcore_barrier()

    pltpu.sync_copy(H_sh.at[pl.ds(sub * RPT, RPT)],
                    Hout.at[c, pl.ds(sub * RPT, RPT)])

    pltpu.sync_copy(d_sh.at[pl.ds(sub * RPT, RPT)],
                    dout.at[c, pl.ds(sub * RPT, RPT)])


_agg_pass = pl.kernel(
    _agg_body,
    out_type=(
        jax.ShapeDtypeStruct((NC, NP, D), _f32),
        jax.ShapeDtypeStruct((NC, NP), _f32),
    ),
    mesh=plsc.VectorSubcoreMesh(core_axis_name="c", subcore_axis_name="s",
                                num_cores=NC, num_subcores=NS),
    compiler_params=pltpu.CompilerParams(needs_layout_passes=False),
    scratch_types=[
        pltpu.VMEM((2, NCH, KE), jnp.int32),
        pltpu.VMEM((2, NCH, KE), jnp.int32),
        pltpu.VMEM((2, NCH, KE), _f32),
        pltpu.VMEM((GS, KE, D), _f32),
        pltpu.VMEM((RPT,), _f32),
        pltpu.VMEM_SHARED((NP, D), _f32),
        pltpu.VMEM_SHARED((NP,), _f32),
        pltpu.SemaphoreType.DMA((GS,)),
        pltpu.SemaphoreType.DMA((GS,)),
    ],
)


def _edge_pass(h, s, t, smax, srcp, dstp):
    w = _wts_pass(s, t, smax, srcp, dstp)
    return _agg_pass(h, srcp, dstp, w)


def kernel(x, edge_index, W1, b1, W2, b2, a):
    src = edge_index[0]
    dst = edge_index[1]
    pad = EPAD - E
    fill = (jnp.arange(pad, dtype=jnp.int32) % N)
    srcp = jnp.concatenate([src, fill]).reshape(NW, ND, KE)
    dstp = jnp.concatenate([dst, fill]).reshape(NW, ND, KE)

    h1, s1, t1, smax1 = _dense1(x, W1, b1, a)
    H1, d1 = _edge_pass(h1, s1, t1, smax1, srcp, dstp)
    h1p, h2, s2, t2, smax2 = _combine_dense(H1, d1, W2, b2, a)
    H2, d2 = _edge_pass(h2, s2, t2, smax2, srcp, dstp)
    return _final(h1p, H2, d2)


# speedup vs baseline: 1.3399x; 1.3399x over previous
import functools

import jax
import jax.numpy as jnp
from jax import lax
from jax.experimental import pallas as pl
from jax.experimental.pallas import tpu as pltpu
from jax.experimental.pallas import tpu_sc as plsc

N = 10000
D = 128
E = 320000
NC = 2
NS = 16
NW = NC * NS
KE = 64
ND = 160
NCH = 4
EPW = ND * KE
EPAD = NW * EPW
NP = 10240
RPT = NP // NS
SLOPE = 0.2


def _leaky(v):
    return jnp.where(v > 0, v, SLOPE * v)


def _dense1_body(x_ref, W_ref, b_ref, a_ref, h_ref, s_ref, t_ref, smax_ref):
    h = jnp.dot(x_ref[...], W_ref[...], preferred_element_type=jnp.float32)
    h = _leaky(h + b_ref[...][None, :])
    h_ref[...] = h
    av = a_ref[...][:, 0]
    s = jnp.sum(h * av[:D][None, :], axis=1)
    t = jnp.sum(h * av[D:][None, :], axis=1)
    s_ref[...] = s
    t_ref[...] = t
    smax_ref[...] = jnp.broadcast_to(jnp.max(s), (16,))


def _combine_dense_body(Hp_ref, dp_ref, W_ref, b_ref, a_ref,
                        hp_ref, h_ref, s_ref, t_ref, smax_ref):
    Hs = Hp_ref[0, :N, :] + Hp_ref[1, :N, :]
    dn = dp_ref[0, :N] + dp_ref[1, :N]
    dn = jnp.where(dn == 0, 1.0, dn)
    hp = Hs / dn[:, None]
    hp_ref[...] = hp
    h = jnp.dot(hp, W_ref[...], preferred_element_type=jnp.float32)
    h = _leaky(h + b_ref[...][None, :])
    h_ref[...] = h
    av = a_ref[...][:, 0]
    s = jnp.sum(h * av[:D][None, :], axis=1)
    t = jnp.sum(h * av[D:][None, :], axis=1)
    s_ref[...] = s
    t_ref[...] = t
    smax_ref[...] = jnp.broadcast_to(jnp.max(s), (16,))


def _final_body(h1p_ref, Hp_ref, dp_ref, out_ref):
    Hs = Hp_ref[0, :N, :] + Hp_ref[1, :N, :]
    dn = dp_ref[0, :N] + dp_ref[1, :N]
    dn = jnp.where(dn == 0, 1.0, dn)
    out_ref[:, :D] = h1p_ref[...]
    out_ref[:, D:] = Hs / dn[:, None]


_f32 = jnp.float32


def _dense1(x, W1, b1, a):
    return pl.pallas_call(
        _dense1_body,
        out_shape=(
            jax.ShapeDtypeStruct((N, D), _f32),
            jax.ShapeDtypeStruct((N,), _f32),
            jax.ShapeDtypeStruct((N,), _f32),
            jax.ShapeDtypeStruct((16,), _f32),
        ),
    )(x, W1, b1, a)


def _combine_dense(Hp, dp, W2, b2, a):
    return pl.pallas_call(
        _combine_dense_body,
        out_shape=(
            jax.ShapeDtypeStruct((N, D), _f32),
            jax.ShapeDtypeStruct((N, D), _f32),
            jax.ShapeDtypeStruct((N,), _f32),
            jax.ShapeDtypeStruct((N,), _f32),
            jax.ShapeDtypeStruct((16,), _f32),
        ),
    )(Hp, dp, W2, b2, a)


def _final(h1p, Hp, dp):
    return pl.pallas_call(
        _final_body,
        out_shape=jax.ShapeDtypeStruct((N, 2 * D), _f32),
    )(h1p, Hp, dp)


def _edge_body(h_hbm, s_hbm, t_hbm, smax_hbm, src_hbm, dst_hbm,
               Hout, dout,
               s_v, t_v, smax_v, src_v, dst_v, w_v, rows_v,
               H_sh, d_sh, sem_g, sem_s):
    c = lax.axis_index("c")
    sub = lax.axis_index("s")
    wid = c * NS + sub

    z16 = jnp.zeros((16,), _f32)

    def _zb(i, _):
        r = i // 8
        q = i % 8
        rows_v[0, r, pl.ds(q * 16, 16)] = z16
        return 0
    lax.fori_loop(0, KE * 8, _zb, 0)

    def _zw(i, _):
        w_v[0, pl.ds(i * 16, 16)] = z16
        return 0
    lax.fori_loop(0, KE // 16, _zw, 0)

    def _zH(i, _):
        pltpu.sync_copy(rows_v.at[0], H_sh.at[pl.ds(sub * RPT + i * KE, KE)])
        pltpu.sync_copy(w_v.at[0], d_sh.at[pl.ds(sub * RPT + i * KE, KE)])
        return 0
    lax.fori_loop(0, RPT // KE, _zH, 0)

    pltpu.sync_copy(s_hbm, s_v)
    pltpu.sync_copy(t_hbm, t_v)
    pltpu.sync_copy(smax_hbm, smax_v)

    plsc.subcore_barrier()

    smax = smax_v[...]
    base0 = wid * EPW

    pltpu.sync_copy(src_hbm.at[wid, pl.ds(0, NCH)], src_v.at[0])
    pltpu.sync_copy(dst_hbm.at[wid, pl.ds(0, NCH)], dst_v.at[0])
    pltpu.async_copy(h_hbm.at[src_v.at[0, 0]], rows_v.at[0], sem_g.at[0])
    pltpu.async_copy(h_hbm.at[src_v.at[0, 1]], rows_v.at[1], sem_g.at[1])

    def _step(j, _):
        p = j % 3
        csl = (j // NCH) % 2
        jj = j % NCH

        pltpu.make_async_copy(h_hbm.at[src_v.at[csl, jj]], rows_v.at[p],
                              sem_g.at[p]).wait()

        valid = jnp.where(base0 + j * KE < E, 1.0, 0.0).astype(_f32)
        for g in range(KE // 16):
            sidx = src_v[csl, jj, pl.ds(g * 16, 16)]
            didx = dst_v[csl, jj, pl.ds(g * 16, 16)]
            sg = plsc.load_gather(s_v, [sidx])
            tg = plsc.load_gather(t_v, [didx])
            e = _leaky(sg + tg)
            cc = _leaky(smax + tg)
            wg = jnp.exp(e - cc) * valid
            w_v[p, pl.ds(g * 16, 16)] = wg
            for r in range(16):
                row = g * 16 + r
                spl = jnp.take_along_axis(
                    wg, jnp.full((16,), r, jnp.int32),
                    axis=0, mode="promise_in_bounds")
                for q in range(D // 16):
                    rows_v[p, row, pl.ds(q * 16, 16)] = (
                        rows_v[p, row, pl.ds(q * 16, 16)] * spl)

        @pl.when(j >= 1)
        def _():
            jm = j - 1
            pm = jm % 3
            cslm = (jm // NCH) % 2
            jjm = jm % NCH
            pltpu.make_async_copy(rows_v.at[pm],
                                  H_sh.at[dst_v.at[cslm, jjm]],
                                  sem_s.at[pm]).wait()
            pltpu.make_async_copy(w_v.at[pm],
                                  d_sh.at[dst_v.at[cslm, jjm]],
                                  sem_s.at[pm]).wait()

        pltpu.async_copy(rows_v.at[p], H_sh.at[dst_v.at[csl, jj]],
                         sem_s.at[p], add=True)
        pltpu.async_copy(w_v.at[p], d_sh.at[dst_v.at[csl, jj]],
                         sem_s.at[p], add=True)

        @pl.when(j + 2 < ND)
        def _():
            j2 = j + 2
            p2 = j2 % 3
            ch2 = j2 // NCH
            csl2 = ch2 % 2
            jj2 = j2 % NCH

            @pl.when(jj2 == 0)
            def _():
                pltpu.sync_copy(src_hbm.at[wid, pl.ds(ch2 * NCH, NCH)],
                                src_v.at[csl2])
                pltpu.sync_copy(dst_hbm.at[wid, pl.ds(ch2 * NCH, NCH)],
                                dst_v.at[csl2])

            pltpu.async_copy(h_hbm.at[src_v.at[csl2, jj2]], rows_v.at[p2],
                             sem_g.at[p2])
        return 0

    lax.fori_loop(0, ND, _step, 0)

    pf = (ND - 1) % 3
    cslf = ((ND - 1) // NCH) % 2
    jjf = (ND - 1) % NCH
    pltpu.make_async_copy(rows_v.at[pf], H_sh.at[dst_v.at[cslf, jjf]],
                          sem_s.at[pf]).wait()
    pltpu.make_async_copy(w_v.at[pf], d_sh.at[dst_v.at[cslf, jjf]],
                          sem_s.at[pf]).wait()

    plsc.subcore_barrier()

    pltpu.sync_copy(H_sh.at[pl.ds(sub * RPT, RPT)],
                    Hout.at[c, pl.ds(sub * RPT, RPT)])

    pltpu.sync_copy(d_sh.at[pl.ds(sub * RPT, RPT)],
                    dout.at[c, pl.ds(sub * RPT, RPT)])


_edge_pass = pl.kernel(
    _edge_body,
    out_type=(
        jax.ShapeDtypeStruct((NC, NP, D), _f32),
        jax.ShapeDtypeStruct((NC, NP), _f32),
    ),
    mesh=plsc.VectorSubcoreMesh(core_axis_name="c", subcore_axis_name="s",
                                num_cores=NC, num_subcores=NS),
    compiler_params=pltpu.CompilerParams(needs_layout_passes=False),
    scratch_types=[
        pltpu.VMEM((N,), _f32),
        pltpu.VMEM((N,), _f32),
        pltpu.VMEM((16,), _f32),
        pltpu.VMEM((2, NCH, KE), jnp.int32),
        pltpu.VMEM((2, NCH, KE), jnp.int32),
        pltpu.VMEM((3, KE), _f32),
        pltpu.VMEM((3, KE, D), _f32),
        pltpu.VMEM_SHARED((NP, D), _f32),
        pltpu.VMEM_SHARED((NP,), _f32),
        pltpu.SemaphoreType.DMA((3,)),
        pltpu.SemaphoreType.DMA((3,)),
    ],
)


def kernel(x, edge_index, W1, b1, W2, b2, a):
    src = edge_index[0]
    dst = edge_index[1]
    pad = EPAD - E
    fill = (jnp.arange(pad, dtype=jnp.int32) % N)
    srcp = jnp.concatenate([src, fill]).reshape(NW, ND, KE)
    dstp = jnp.concatenate([dst, fill]).reshape(NW, ND, KE)

    h1, s1, t1, smax1 = _dense1(x, W1, b1, a)
    H1, d1 = _edge_pass(h1, s1, t1, smax1, srcp, dstp)
    h1p, h2, s2, t2, smax2 = _combine_dense(H1, d1, W2, b2, a)
    H2, d2 = _edge_pass(h2, s2, t2, smax2, srcp, dstp)
    return _final(h1p, H2, d2)


# speedup vs baseline: 1.4499x; 1.0822x over previous
import functools

import jax
import jax.numpy as jnp
from jax import lax
from jax.experimental import pallas as pl
from jax.experimental.pallas import tpu as pltpu
from jax.experimental.pallas import tpu_sc as plsc

N = 10000
D = 128
E = 320000
NC = 2
NS = 16
NW = NC * NS
KE = 64
ND = 160
NCH = 4
EPW = ND * KE
EPAD = NW * EPW
NP = 10240
RPT = NP // NS
SLOPE = 0.2


def _leaky(v):
    return jnp.where(v > 0, v, SLOPE * v)


def _dense1_body(x_ref, W_ref, b_ref, a_ref, h_ref, s_ref, t_ref, smax_ref):
    h = jnp.dot(x_ref[...], W_ref[...], preferred_element_type=jnp.float32)
    h = _leaky(h + b_ref[...][None, :])
    h_ref[...] = h
    av = a_ref[...][:, 0]
    s = jnp.sum(h * av[:D][None, :], axis=1)
    t = jnp.sum(h * av[D:][None, :], axis=1)
    s_ref[...] = s
    t_ref[...] = t
    smax_ref[...] = jnp.broadcast_to(jnp.max(s), (16,))


def _combine_dense_body(Hp_ref, dp_ref, W_ref, b_ref, a_ref,
                        hp_ref, h_ref, s_ref, t_ref, smax_ref):
    Hs = Hp_ref[0, :N, :] + Hp_ref[1, :N, :]
    dn = dp_ref[0, :N] + dp_ref[1, :N]
    dn = jnp.where(dn == 0, 1.0, dn)
    hp = Hs / dn[:, None]
    hp_ref[...] = hp
    h = jnp.dot(hp, W_ref[...], preferred_element_type=jnp.float32)
    h = _leaky(h + b_ref[...][None, :])
    h_ref[...] = h
    av = a_ref[...][:, 0]
    s = jnp.sum(h * av[:D][None, :], axis=1)
    t = jnp.sum(h * av[D:][None, :], axis=1)
    s_ref[...] = s
    t_ref[...] = t
    smax_ref[...] = jnp.broadcast_to(jnp.max(s), (16,))


def _final_body(h1p_ref, Hp_ref, dp_ref, out_ref):
    Hs = Hp_ref[0, :N, :] + Hp_ref[1, :N, :]
    dn = dp_ref[0, :N] + dp_ref[1, :N]
    dn = jnp.where(dn == 0, 1.0, dn)
    out_ref[:, :D] = h1p_ref[...]
    out_ref[:, D:] = Hs / dn[:, None]


_f32 = jnp.float32


def _dense1(x, W1, b1, a):
    return pl.pallas_call(
        _dense1_body,
        out_shape=(
            jax.ShapeDtypeStruct((N, D), _f32),
            jax.ShapeDtypeStruct((N,), _f32),
            jax.ShapeDtypeStruct((N,), _f32),
            jax.ShapeDtypeStruct((16,), _f32),
        ),
    )(x, W1, b1, a)


def _combine_dense(Hp, dp, W2, b2, a):
    return pl.pallas_call(
        _combine_dense_body,
        out_shape=(
            jax.ShapeDtypeStruct((N, D), _f32),
            jax.ShapeDtypeStruct((N, D), _f32),
            jax.ShapeDtypeStruct((N,), _f32),
            jax.ShapeDtypeStruct((N,), _f32),
            jax.ShapeDtypeStruct((16,), _f32),
        ),
    )(Hp, dp, W2, b2, a)


def _final(h1p, Hp, dp):
    return pl.pallas_call(
        _final_body,
        out_shape=jax.ShapeDtypeStruct((N, 2 * D), _f32),
    )(h1p, Hp, dp)


def _edge_body(h_hbm, s_hbm, t_hbm, smax_hbm, esd_hbm,
               Hout, dout,
               s_v, t_v, smax_v, sd_v, w_v, rows_v,
               H_sh, d_sh, sem_g, sem_s):
    c = lax.axis_index("c")
    sub = lax.axis_index("s")
    wid = c * NS + sub

    z16 = jnp.zeros((16,), _f32)

    def _zb(i, _):
        r = i // 8
        q = i % 8
        rows_v[0, r, pl.ds(q * 16, 16)] = z16
        return 0
    lax.fori_loop(0, KE * 8, _zb, 0)

    def _zw(i, _):
        w_v[0, pl.ds(i * 16, 16)] = z16
        return 0
    lax.fori_loop(0, KE // 16, _zw, 0)

    def _zH(i, _):
        pltpu.sync_copy(rows_v.at[0], H_sh.at[pl.ds(sub * RPT + i * KE, KE)])
        pltpu.sync_copy(w_v.at[0], d_sh.at[pl.ds(sub * RPT + i * KE, KE)])
        return 0
    lax.fori_loop(0, RPT // KE, _zH, 0)

    pltpu.sync_copy(s_hbm, s_v)
    pltpu.sync_copy(t_hbm, t_v)
    pltpu.sync_copy(smax_hbm, smax_v)

    plsc.subcore_barrier()

    smax = smax_v[...]
    base0 = wid * EPW

    pltpu.sync_copy(esd_hbm.at[wid, 0], sd_v.at[0])
    pltpu.async_copy(h_hbm.at[sd_v.at[0, 0]], rows_v.at[0], sem_g.at[0])
    pltpu.async_copy(h_hbm.at[sd_v.at[0, 1]], rows_v.at[1], sem_g.at[1])

    def _step(j, _):
        p = j % 3
        csl = (j // NCH) % 2
        jj = j % NCH

        pltpu.make_async_copy(h_hbm.at[sd_v.at[csl, jj]], rows_v.at[p],
                              sem_g.at[p]).wait()

        valid = jnp.where(base0 + j * KE < E, 1.0, 0.0).astype(_f32)
        for g in range(KE // 16):
            sidx = sd_v[csl, jj, pl.ds(g * 16, 16)]
            didx = sd_v[csl, NCH + jj, pl.ds(g * 16, 16)]
            sg = plsc.load_gather(s_v, [sidx])
            tg = plsc.load_gather(t_v, [didx])
            e = _leaky(sg + tg)
            cc = _leaky(smax + tg)
            wg = jnp.exp(e - cc) * valid
            w_v[p, pl.ds(g * 16, 16)] = wg
            for r in range(16):
                row = g * 16 + r
                spl = jnp.take_along_axis(
                    wg, jnp.full((16,), r, jnp.int32),
                    axis=0, mode="promise_in_bounds")
                for q in range(D // 16):
                    rows_v[p, row, pl.ds(q * 16, 16)] = (
                        rows_v[p, row, pl.ds(q * 16, 16)] * spl)

        @pl.when(j >= 1)
        def _():
            jm = j - 1
            pm = jm % 3
            cslm = (jm // NCH) % 2
            jjm = jm % NCH
            pltpu.make_async_copy(rows_v.at[pm],
                                  H_sh.at[sd_v.at[cslm, NCH + jjm]],
                                  sem_s.at[pm]).wait()
            pltpu.make_async_copy(w_v.at[pm],
                                  d_sh.at[sd_v.at[cslm, NCH + jjm]],
                                  sem_s.at[pm]).wait()

        pltpu.async_copy(rows_v.at[p], H_sh.at[sd_v.at[csl, NCH + jj]],
                         sem_s.at[p], add=True)
        pltpu.async_copy(w_v.at[p], d_sh.at[sd_v.at[csl, NCH + jj]],
                         sem_s.at[p], add=True)

        @pl.when(j + 2 < ND)
        def _():
            j2 = j + 2
            p2 = j2 % 3
            ch2 = j2 // NCH
            csl2 = ch2 % 2
            jj2 = j2 % NCH

            @pl.when(jj2 == 0)
            def _():
                pltpu.sync_copy(esd_hbm.at[wid, ch2], sd_v.at[csl2])

            pltpu.async_copy(h_hbm.at[sd_v.at[csl2, jj2]], rows_v.at[p2],
                             sem_g.at[p2])
        return 0

    lax.fori_loop(0, ND, _step, 0)

    pf = (ND - 1) % 3
    cslf = ((ND - 1) // NCH) % 2
    jjf = (ND - 1) % NCH
    pltpu.make_async_copy(rows_v.at[pf], H_sh.at[sd_v.at[cslf, NCH + jjf]],
                          sem_s.at[pf]).wait()
    pltpu.make_async_copy(w_v.at[pf], d_sh.at[sd_v.at[cslf, NCH + jjf]],
                          sem_s.at[pf]).wait()

    plsc.subcore_barrier()

    pltpu.sync_copy(H_sh.at[pl.ds(sub * RPT, RPT)],
                    Hout.at[c, pl.ds(sub * RPT, RPT)])

    pltpu.sync_copy(d_sh.at[pl.ds(sub * RPT, RPT)],
                    dout.at[c, pl.ds(sub * RPT, RPT)])


_edge_pass = pl.kernel(
    _edge_body,
    out_type=(
        jax.ShapeDtypeStruct((NC, NP, D), _f32),
        jax.ShapeDtypeStruct((NC, NP), _f32),
    ),
    mesh=plsc.VectorSubcoreMesh(core_axis_name="c", subcore_axis_name="s",
                                num_cores=NC, num_subcores=NS),
    compiler_params=pltpu.CompilerParams(needs_layout_passes=False),
    scratch_types=[
        pltpu.VMEM((N,), _f32),
        pltpu.VMEM((N,), _f32),
        pltpu.VMEM((16,), _f32),
        pltpu.VMEM((2, 2 * NCH, KE), jnp.int32),
        pltpu.VMEM((3, KE), _f32),
        pltpu.VMEM((3, KE, D), _f32),
        pltpu.VMEM_SHARED((NP, D), _f32),
        pltpu.VMEM_SHARED((NP,), _f32),
        pltpu.SemaphoreType.DMA((3,)),
        pltpu.SemaphoreType.DMA((3,)),
    ],
)


def kernel(x, edge_index, W1, b1, W2, b2, a):
    src = edge_index[0]
    dst = edge_index[1]
    pad = EPAD - E
    fill = (jnp.arange(pad, dtype=jnp.int32) % N)
    srcp = jnp.concatenate([src, fill]).reshape(NW, ND // NCH, NCH, KE)
    dstp = jnp.concatenate([dst, fill]).reshape(NW, ND // NCH, NCH, KE)
    esd = jnp.concatenate([srcp, dstp], axis=2)

    h1, s1, t1, smax1 = _dense1(x, W1, b1, a)
    H1, d1 = _edge_pass(h1, s1, t1, smax1, esd)
    h1p, h2, s2, t2, smax2 = _combine_dense(H1, d1, W2, b2, a)
    H2, d2 = _edge_pass(h2, s2, t2, smax2, esd)
    return _final(h1p, H2, d2)


# speedup vs baseline: 1.5899x; 1.0965x over previous
import functools

import jax
import jax.numpy as jnp
from jax import lax
from jax.experimental import pallas as pl
from jax.experimental.pallas import tpu as pltpu
from jax.experimental.pallas import tpu_sc as plsc

N = 10000
D = 128
E = 320000
NC = 2
NS = 16
NW = NC * NS
KE = 64
ND = 160
NCH = 4
EPW = ND * KE
EPAD = NW * EPW
NP = 10240
RPT = NP // NS
SLOPE = 0.2


def _leaky(v):
    return jnp.where(v > 0, v, SLOPE * v)


def _dense1_body(x_ref, W_ref, b_ref, a_ref, h_ref, s_ref, t_ref, smax_ref):
    h = jnp.dot(x_ref[...], W_ref[...], preferred_element_type=jnp.float32)
    h = _leaky(h + b_ref[...][None, :])
    h_ref[...] = h
    av = a_ref[...][:, 0]
    s = jnp.sum(h * av[:D][None, :], axis=1)
    t = jnp.sum(h * av[D:][None, :], axis=1)
    s_ref[...] = s
    t_ref[...] = t
    smax_ref[...] = jnp.broadcast_to(jnp.max(s), (16,))


def _combine_dense_body(Hp_ref, dp_ref, W_ref, b_ref, a_ref,
                        hp_ref, h_ref, s_ref, t_ref, smax_ref):
    Hs = Hp_ref[0, :N, :] + Hp_ref[1, :N, :]
    dn = dp_ref[0, :N] + dp_ref[1, :N]
    dn = jnp.where(dn == 0, 1.0, dn)
    hp = Hs / dn[:, None]
    hp_ref[...] = hp
    h = jnp.dot(hp, W_ref[...], preferred_element_type=jnp.float32)
    h = _leaky(h + b_ref[...][None, :])
    h_ref[...] = h
    av = a_ref[...][:, 0]
    s = jnp.sum(h * av[:D][None, :], axis=1)
    t = jnp.sum(h * av[D:][None, :], axis=1)
    s_ref[...] = s
    t_ref[...] = t
    smax_ref[...] = jnp.broadcast_to(jnp.max(s), (16,))


def _final_body(h1p_ref, Hp_ref, dp_ref, out_ref):
    Hs = Hp_ref[0, :N, :] + Hp_ref[1, :N, :]
    dn = dp_ref[0, :N] + dp_ref[1, :N]
    dn = jnp.where(dn == 0, 1.0, dn)
    out_ref[:, :D] = h1p_ref[...]
    out_ref[:, D:] = Hs / dn[:, None]


_f32 = jnp.float32


def _dense1(x, W1, b1, a):
    return pl.pallas_call(
        _dense1_body,
        out_shape=(
            jax.ShapeDtypeStruct((N, D), _f32),
            jax.ShapeDtypeStruct((N,), _f32),
            jax.ShapeDtypeStruct((N,), _f32),
            jax.ShapeDtypeStruct((16,), _f32),
        ),
    )(x, W1, b1, a)


def _combine_dense(Hp, dp, W2, b2, a):
    return pl.pallas_call(
        _combine_dense_body,
        out_shape=(
            jax.ShapeDtypeStruct((N, D), _f32),
            jax.ShapeDtypeStruct((N, D), _f32),
            jax.ShapeDtypeStruct((N,), _f32),
            jax.ShapeDtypeStruct((N,), _f32),
            jax.ShapeDtypeStruct((16,), _f32),
        ),
    )(Hp, dp, W2, b2, a)


def _final(h1p, Hp, dp):
    return pl.pallas_call(
        _final_body,
        out_shape=jax.ShapeDtypeStruct((N, 2 * D), _f32),
    )(h1p, Hp, dp)


def _edge_body(h_hbm, s_hbm, t_hbm, smax_hbm, esd_hbm,
               Hout, dout,
               s_v, t_v, smax_v, sd_v, w_v, rows_v,
               H_sh, d_sh, sem_g, sem_s, sem_i):
    c = lax.axis_index("c")
    sub = lax.axis_index("s")
    wid = c * NS + sub

    z16 = jnp.zeros((16,), _f32)

    def _zb(i, _):
        r = i // 8
        q = i % 8
        rows_v[0, r, pl.ds(q * 16, 16)] = z16
        return 0
    lax.fori_loop(0, KE * 8, _zb, 0)

    def _zw(i, _):
        w_v[0, pl.ds(i * 16, 16)] = z16
        return 0
    lax.fori_loop(0, KE // 16, _zw, 0)

    def _zH(i, _):
        pltpu.sync_copy(rows_v.at[0], H_sh.at[pl.ds(sub * RPT + i * KE, KE)])
        pltpu.sync_copy(w_v.at[0], d_sh.at[pl.ds(sub * RPT + i * KE, KE)])
        return 0
    lax.fori_loop(0, RPT // KE, _zH, 0)

    pltpu.sync_copy(s_hbm, s_v)
    pltpu.sync_copy(t_hbm, t_v)
    pltpu.sync_copy(smax_hbm, smax_v)

    plsc.subcore_barrier()

    smax = smax_v[...]
    base0 = wid * EPW

    pltpu.sync_copy(esd_hbm.at[wid, 0], sd_v.at[0])
    pltpu.async_copy(esd_hbm.at[wid, 1], sd_v.at[1], sem_i.at[1])
    pltpu.async_copy(h_hbm.at[sd_v.at[0, 0]], rows_v.at[0], sem_g.at[0])
    pltpu.async_copy(h_hbm.at[sd_v.at[0, 1]], rows_v.at[1], sem_g.at[1])

    def _step(j, _):
        p = j % 3
        csl = (j // NCH) % 3
        jj = j % NCH

        pltpu.make_async_copy(h_hbm.at[sd_v.at[csl, jj]], rows_v.at[p],
                              sem_g.at[p]).wait()

        valid = jnp.where(base0 + j * KE < E, 1.0, 0.0).astype(_f32)
        for g in range(KE // 16):
            sidx = sd_v[csl, jj, pl.ds(g * 16, 16)]
            didx = sd_v[csl, NCH + jj, pl.ds(g * 16, 16)]
            sg = plsc.load_gather(s_v, [sidx])
            tg = plsc.load_gather(t_v, [didx])
            e = _leaky(sg + tg)
            cc = _leaky(smax + tg)
            wg = jnp.exp(e - cc) * valid
            w_v[p, pl.ds(g * 16, 16)] = wg
            for r in range(16):
                row = g * 16 + r
                spl = jnp.take_along_axis(
                    wg, jnp.full((16,), r, jnp.int32),
                    axis=0, mode="promise_in_bounds")
                for q in range(D // 16):
                    rows_v[p, row, pl.ds(q * 16, 16)] = (
                        rows_v[p, row, pl.ds(q * 16, 16)] * spl)

        @pl.when(j >= 1)
        def _():
            jm = j - 1
            pm = jm % 3
            cslm = (jm // NCH) % 3
            jjm = jm % NCH
            pltpu.make_async_copy(rows_v.at[pm],
                                  H_sh.at[sd_v.at[cslm, NCH + jjm]],
                                  sem_s.at[pm]).wait()
            pltpu.make_async_copy(w_v.at[pm],
                                  d_sh.at[sd_v.at[cslm, NCH + jjm]],
                                  sem_s.at[pm]).wait()

        pltpu.async_copy(rows_v.at[p], H_sh.at[sd_v.at[csl, NCH + jj]],
                         sem_s.at[p], add=True)
        pltpu.async_copy(w_v.at[p], d_sh.at[sd_v.at[csl, NCH + jj]],
                         sem_s.at[p], add=True)

        @pl.when(j + 2 < ND)
        def _():
            j2 = j + 2
            p2 = j2 % 3
            ch2 = j2 // NCH
            csl2 = ch2 % 3
            jj2 = j2 % NCH

            @pl.when(jj2 == 0)
            def _():
                pltpu.make_async_copy(esd_hbm.at[wid, ch2], sd_v.at[csl2],
                                      sem_i.at[csl2]).wait()

                @pl.when(ch2 + 1 < ND // NCH)
                def _():
                    pltpu.async_copy(esd_hbm.at[wid, ch2 + 1],
                                     sd_v.at[(ch2 + 1) % 3],
                                     sem_i.at[(ch2 + 1) % 3])

            pltpu.async_copy(h_hbm.at[sd_v.at[csl2, jj2]], rows_v.at[p2],
                             sem_g.at[p2])
        return 0

    lax.fori_loop(0, ND, _step, 0)

    pf = (ND - 1) % 3
    cslf = ((ND - 1) // NCH) % 3
    jjf = (ND - 1) % NCH
    pltpu.make_async_copy(rows_v.at[pf], H_sh.at[sd_v.at[cslf, NCH + jjf]],
                          sem_s.at[pf]).wait()
    pltpu.make_async_copy(w_v.at[pf], d_sh.at[sd_v.at[cslf, NCH + jjf]],
                          sem_s.at[pf]).wait()

    plsc.subcore_barrier()

    pltpu.sync_copy(H_sh.at[pl.ds(sub * RPT, RPT)],
                    Hout.at[c, pl.ds(sub * RPT, RPT)])

    pltpu.sync_copy(d_sh.at[pl.ds(sub * RPT, RPT)],
                    dout.at[c, pl.ds(sub * RPT, RPT)])


_edge_pass = pl.kernel(
    _edge_body,
    out_type=(
        jax.ShapeDtypeStruct((NC, NP, D), _f32),
        jax.ShapeDtypeStruct((NC, NP), _f32),
    ),
    mesh=plsc.VectorSubcoreMesh(core_axis_name="c", subcore_axis_name="s",
                                num_cores=NC, num_subcores=NS),
    compiler_params=pltpu.CompilerParams(needs_layout_passes=False),
    scratch_types=[
        pltpu.VMEM((N,), _f32),
        pltpu.VMEM((N,), _f32),
        pltpu.VMEM((16,), _f32),
        pltpu.VMEM((3, 2 * NCH, KE), jnp.int32),
        pltpu.VMEM((3, KE), _f32),
        pltpu.VMEM((3, KE, D), _f32),
        pltpu.VMEM_SHARED((NP, D), _f32),
        pltpu.VMEM_SHARED((NP,), _f32),
        pltpu.SemaphoreType.DMA((3,)),
        pltpu.SemaphoreType.DMA((3,)),
        pltpu.SemaphoreType.DMA((3,)),
    ],
)


def kernel(x, edge_index, W1, b1, W2, b2, a):
    src = edge_index[0]
    dst = edge_index[1]
    pad = EPAD - E
    fill = (jnp.arange(pad, dtype=jnp.int32) % N)
    srcp = jnp.concatenate([src, fill]).reshape(NW, ND // NCH, NCH, KE)
    dstp = jnp.concatenate([dst, fill]).reshape(NW, ND // NCH, NCH, KE)
    esd = jnp.concatenate([srcp, dstp], axis=2)

    h1, s1, t1, smax1 = _dense1(x, W1, b1, a)
    H1, d1 = _edge_pass(h1, s1, t1, smax1, esd)
    h1p, h2, s2, t2, smax2 = _combine_dense(H1, d1, W2, b2, a)
    H2, d2 = _edge_pass(h2, s2, t2, smax2, esd)
    return _final(h1p, H2, d2)
